# baseline (speedup 1.0000x reference)
import functools

import jax
import jax.numpy as jnp
from jax import lax
from jax.experimental import pallas as pl
from jax.experimental.pallas import tpu as pltpu
from jax.experimental.pallas import tpu_sc as plsc

_INFO = plsc.get_sparse_core_info()
_NC, _NS = _INFO.num_cores, _INFO.num_subcores
_NW = _NC * _NS

_N = 4 * 4096
_D = 1024
_RPW = _N // _NW
_CH = 32
_NCHUNK = _RPW // _CH

_mesh = plsc.VectorSubcoreMesh(core_axis_name="c", subcore_axis_name="s")


@functools.partial(
    pl.kernel,
    mesh=_mesh,
    out_type=jax.ShapeDtypeStruct((_N, _D), jnp.float32),
    scratch_types=[
        pltpu.VMEM((_RPW,), jnp.int32),
        pltpu.VMEM((_CH, _D), jnp.float32),
        pltpu.VMEM((_CH, _D), jnp.float32),
        pltpu.VMEM((_CH, _D), jnp.float32),
        pltpu.SemaphoreType.DMA,
        pltpu.SemaphoreType.DMA,
    ],
)
def _gather_rows(table_hbm, idx_hbm, out_hbm, idx_v, buf0, buf1, buf2, gsem, osem):
    wid = lax.axis_index("s") * _NC + lax.axis_index("c")
    base = wid * _RPW
    pltpu.sync_copy(idx_hbm.at[pl.ds(base, _RPW)], idx_v)
    bufs = (buf0, buf1, buf2)
    nbuf = len(bufs)

    gd = [None] * _NCHUNK
    od = [None] * _NCHUNK
    for c in range(_NCHUNK):
        b = c % nbuf
        if c >= nbuf:
            od[c - nbuf].wait()
        gd[c] = pltpu.async_copy(
            table_hbm.at[idx_v.at[pl.ds(c * _CH, _CH)]], bufs[b], gsem
        )
        if c >= 1:
            pb = (c - 1) % nbuf
            gd[c - 1].wait()
            od[c - 1] = pltpu.async_copy(
                bufs[pb], out_hbm.at[pl.ds(base + (c - 1) * _CH, _CH)], osem
            )
    last = _NCHUNK - 1
    gd[last].wait()
    od[last] = pltpu.async_copy(
        bufs[last % nbuf], out_hbm.at[pl.ds(base + last * _CH, _CH)], osem
    )
    od[last - 1].wait()
    od[last].wait()


def kernel(pos_idx, time, pos_emb):
    del time
    idx = pos_idx.reshape(-1)
    table = pos_emb.reshape(pos_emb.shape[-2], pos_emb.shape[-1])
    out = _gather_rows(table, idx)
    return out.reshape(pos_idx.shape + (pos_emb.shape[-1],))


# speedup vs baseline: 1.0090x; 1.0090x over previous
import functools

import jax
import jax.numpy as jnp
from jax import lax
from jax.experimental import pallas as pl
from jax.experimental.pallas import tpu as pltpu
from jax.experimental.pallas import tpu_sc as plsc

_INFO = plsc.get_sparse_core_info()
_NC, _NS = _INFO.num_cores, _INFO.num_subcores
_NW = _NC * _NS

_N = 4 * 4096
_D = 1024
_RPW = _N // _NW
_CH = 32
_NCHUNK = _RPW // _CH

_mesh = plsc.VectorSubcoreMesh(core_axis_name="c", subcore_axis_name="s")


@functools.partial(
    pl.kernel,
    mesh=_mesh,
    out_type=jax.ShapeDtypeStruct((_N, _D), jnp.float32),
    scratch_types=[
        pltpu.VMEM((_RPW,), jnp.int32),
        pltpu.VMEM((_CH, _D), jnp.float32),
        pltpu.VMEM((_CH, _D), jnp.float32),
        pltpu.SemaphoreType.DMA,
        pltpu.SemaphoreType.DMA,
    ],
)
def _gather_rows(table_hbm, idx_hbm, out_hbm, idx_v, buf0, buf1, gsem, osem):
    wid = lax.axis_index("s") * _NC + lax.axis_index("c")
    base = wid * _RPW
    pltpu.sync_copy(idx_hbm.at[pl.ds(base, _RPW)], idx_v)

    def gather(c, buf):
        return pltpu.async_copy(
            table_hbm.at[idx_v.at[pl.ds(c * _CH, _CH)]], buf, gsem
        )

    def copy_out(c, buf):
        return pltpu.async_copy(buf, out_hbm.at[pl.ds(base + c * _CH, _CH)], osem)

    def wait_gather(buf):
        pltpu.make_async_copy(table_hbm.at[idx_v.at[pl.ds(0, _CH)]], buf, gsem).wait()

    def wait_out(buf):
        pltpu.make_async_copy(buf, out_hbm.at[pl.ds(base, _CH)], osem).wait()

    gather(0, buf0)
    gather(1, buf1)

    @pl.loop(0, _NCHUNK // 2 - 1)
    def _pair(k):
        c0 = 2 * k
        wait_gather(buf0)
        copy_out(c0, buf0)
        wait_gather(buf1)
        copy_out(c0 + 1, buf1)
        wait_out(buf0)
        gather(c0 + 2, buf0)
        wait_out(buf1)
        gather(c0 + 3, buf1)

    last = _NCHUNK - 2
    wait_gather(buf0)
    copy_out(last, buf0)
    wait_gather(buf1)
    copy_out(last + 1, buf1)
    wait_out(buf0)
    wait_out(buf1)


def kernel(pos_idx, time, pos_emb):
    del time
    idx = pos_idx.reshape(-1)
    table = pos_emb.reshape(pos_emb.shape[-2], pos_emb.shape[-1])
    out = _gather_rows(table, idx)
    return out.reshape(pos_idx.shape + (pos_emb.shape[-1],))
